# Initial kernel scaffold; baseline (speedup 1.0000x reference)
import jax
import jax.numpy as jnp
from jax.experimental import pallas as pl


def kernel(inputs, emb_weight):
    raise NotImplementedError("write your pallas kernel here")


# speedup vs baseline: 1.3708x; 1.3708x over previous
import functools

import jax
import jax.numpy as jnp
from jax.experimental import pallas as pl
from jax.experimental.pallas import tpu as pltpu

NUM_EMBEDDINGS = 1024
EMBEDDING_DIM = 64
COMMITMENT_COST = 0.25
N_ROWS = 8192
BLOCK_ROWS = 1024
N_BLOCKS = N_ROWS // BLOCK_ROWS


def _vq_kernel(x_ref, emb_ref, enc_ref, q_ref, loss_ref, perp_ref,
               loss_acc, hist_acc):
    step = pl.program_id(0)
    x = x_ref[:]
    emb = emb_ref[:]

    x2 = jnp.sum(x * x, axis=1, keepdims=True)
    e2 = jnp.sum(emb * emb, axis=1)
    m = jax.lax.dot_general(x, emb, (((1,), (1,)), ((), ())),
                            preferred_element_type=jnp.float32)
    d = (x2 + e2[None, :]) - 2.0 * m

    idx = jnp.argmin(d, axis=1)
    iota = jax.lax.broadcasted_iota(jnp.int32, (BLOCK_ROWS, NUM_EMBEDDINGS), 1)
    onehot = (idx[:, None] == iota).astype(jnp.float32)
    enc_ref[:] = onehot

    q = jnp.dot(onehot, emb, preferred_element_type=jnp.float32)
    q_ref[:] = x + (q - x)

    @pl.when(step == 0)
    def _init():
        loss_acc[:] = jnp.zeros_like(loss_acc)
        hist_acc[:] = jnp.zeros_like(hist_acc)

    loss_acc[:] += jnp.sum((q - x) ** 2).reshape(1, 1)
    hist_acc[:] += jnp.sum(onehot, axis=0, keepdims=True)

    @pl.when(step == N_BLOCKS - 1)
    def _fin():
        loss_ref[:] = COMMITMENT_COST * loss_acc[:] / (N_ROWS * EMBEDDING_DIM)
        p = hist_acc[:] / float(N_ROWS)
        perp_ref[:] = jnp.exp(-jnp.sum(p * jnp.log(p + 1e-10))).reshape(1, 1)


@functools.partial(jax.jit, static_argnames=("interpret",))
def kernel(inputs, emb_weight, interpret=False):
    x = jnp.transpose(inputs, (0, 2, 3, 1)).reshape(N_ROWS, EMBEDDING_DIM)

    enc, q_st, loss, perp = pl.pallas_call(
        _vq_kernel,
        grid=(N_BLOCKS,),
        in_specs=[
            pl.BlockSpec((BLOCK_ROWS, EMBEDDING_DIM), lambda i: (i, 0)),
            pl.BlockSpec((NUM_EMBEDDINGS, EMBEDDING_DIM), lambda i: (0, 0)),
        ],
        out_specs=[
            pl.BlockSpec((BLOCK_ROWS, NUM_EMBEDDINGS), lambda i: (i, 0)),
            pl.BlockSpec((BLOCK_ROWS, EMBEDDING_DIM), lambda i: (i, 0)),
            pl.BlockSpec((1, 1), lambda i: (0, 0)),
            pl.BlockSpec((1, 1), lambda i: (0, 0)),
        ],
        out_shape=[
            jax.ShapeDtypeStruct((N_ROWS, NUM_EMBEDDINGS), jnp.float32),
            jax.ShapeDtypeStruct((N_ROWS, EMBEDDING_DIM), jnp.float32),
            jax.ShapeDtypeStruct((1, 1), jnp.float32),
            jax.ShapeDtypeStruct((1, 1), jnp.float32),
        ],
        scratch_shapes=[
            pltpu.VMEM((1, 1), jnp.float32),
            pltpu.VMEM((1, NUM_EMBEDDINGS), jnp.float32),
        ],
        interpret=interpret,
    )(x, emb_weight)

    quantized_out = jnp.transpose(
        q_st.reshape(8, 32, 32, EMBEDDING_DIM), (0, 3, 1, 2))
    return (loss[0, 0], quantized_out, perp[0, 0], enc)
